# Initial kernel scaffold; baseline (speedup 1.0000x reference)
import jax
import jax.numpy as jnp
from jax.experimental import pallas as pl


def kernel(x, edge_index, edge_attr, Wx, bx, We, be):
    raise NotImplementedError("write your pallas kernel here")


# speedup vs baseline: 3.4394x; 3.4394x over previous
import functools

import jax
import jax.numpy as jnp
from jax import lax
from jax.experimental import pallas as pl
from jax.experimental.pallas import tpu as pltpu
from jax.experimental.pallas import tpu_sc as plsc

N_NODES = 10000
N_EDGES = 160000
D_X_IN = 256
D_X_OUT = 256
D_E_IN = 16
D_E_OUT = 64

NUM_SC = 2
NUM_SUBCORES = 16
NUM_WORKERS = NUM_SC * NUM_SUBCORES

EDGES_PER_WORKER = N_EDGES // NUM_WORKERS
CHUNK = 128
FULL_CHUNKS = EDGES_PER_WORKER // CHUNK
TAIL = EDGES_PER_WORKER - FULL_CHUNKS * CHUNK
ROWS_MAIN = 624
ROWS_LAST = N_NODES - (NUM_SUBCORES - 1) * ROWS_MAIN


def _sc_segment_sum(edge_attr, dst, zeros):
    mesh = plsc.VectorSubcoreMesh(core_axis_name="c", subcore_axis_name="s")
    part = jax.ShapeDtypeStruct((N_NODES, D_E_IN), jnp.float32)

    @functools.partial(
        pl.kernel,
        out_type=[part, part],
        mesh=mesh,
        compiler_params=pltpu.CompilerParams(use_tc_tiling_on_sc=False),
        scratch_types=[
            pltpu.VMEM_SHARED((N_NODES, D_E_IN), jnp.float32),
            pltpu.VMEM((CHUNK,), jnp.int32),
            pltpu.VMEM((CHUNK, D_E_IN), jnp.float32),
            pltpu.VMEM((TAIL,), jnp.int32),
            pltpu.VMEM((TAIL, D_E_IN), jnp.float32),
        ],
    )
    def seg_sum(ea_hbm, dst_hbm, zero_hbm, p0_hbm, p1_hbm, acc, idx_v, rows_v,
                idx_t, rows_t):
        cid = lax.axis_index("c")
        sid = lax.axis_index("s")
        wid = sid * NUM_SC + cid
        base = wid * EDGES_PER_WORKER
        row0 = sid * ROWS_MAIN

        @pl.when(sid < NUM_SUBCORES - 1)
        def _():
            pltpu.sync_copy(zero_hbm.at[pl.ds(row0, ROWS_MAIN)],
                            acc.at[pl.ds(row0, ROWS_MAIN)])

        @pl.when(sid == NUM_SUBCORES - 1)
        def _():
            pltpu.sync_copy(zero_hbm.at[pl.ds(row0, ROWS_LAST)],
                            acc.at[pl.ds(row0, ROWS_LAST)])

        plsc.subcore_barrier()

        @pl.loop(0, FULL_CHUNKS)
        def _(j):
            off = base + j * CHUNK
            pltpu.sync_copy(dst_hbm.at[pl.ds(off, CHUNK)], idx_v)
            pltpu.sync_copy(ea_hbm.at[pl.ds(off, CHUNK)], rows_v)
            pltpu.sync_copy(rows_v, acc.at[idx_v], add=True)

        off = base + FULL_CHUNKS * CHUNK
        pltpu.sync_copy(dst_hbm.at[pl.ds(off, TAIL)], idx_t)
        pltpu.sync_copy(ea_hbm.at[pl.ds(off, TAIL)], rows_t)
        pltpu.sync_copy(rows_t, acc.at[idx_t], add=True)

        plsc.subcore_barrier()

        @pl.when(jnp.logical_and(cid == 0, sid < NUM_SUBCORES - 1))
        def _():
            pltpu.sync_copy(acc.at[pl.ds(row0, ROWS_MAIN)],
                            p0_hbm.at[pl.ds(row0, ROWS_MAIN)])

        @pl.when(jnp.logical_and(cid == 0, sid == NUM_SUBCORES - 1))
        def _():
            pltpu.sync_copy(acc.at[pl.ds(row0, ROWS_LAST)],
                            p0_hbm.at[pl.ds(row0, ROWS_LAST)])

        @pl.when(jnp.logical_and(cid == 1, sid < NUM_SUBCORES - 1))
        def _():
            pltpu.sync_copy(acc.at[pl.ds(row0, ROWS_MAIN)],
                            p1_hbm.at[pl.ds(row0, ROWS_MAIN)])

        @pl.when(jnp.logical_and(cid == 1, sid == NUM_SUBCORES - 1))
        def _():
            pltpu.sync_copy(acc.at[pl.ds(row0, ROWS_LAST)],
                            p1_hbm.at[pl.ds(row0, ROWS_LAST)])

    return seg_sum(edge_attr, dst, zeros)


ROW_BLOCK = 1000
GRID_M = N_NODES // ROW_BLOCK


def _tc_body(x_ref, wxt_ref, bx_ref, p0_ref, p1_ref, wet_ref, be_ref, out_ref):
    xh = jnp.dot(x_ref[...], wxt_ref[...],
                 preferred_element_type=jnp.float32,
                 precision=lax.Precision.HIGHEST)
    out_ref[:, :D_X_OUT] = xh + bx_ref[...]
    es = p0_ref[...] + p1_ref[...]
    es = jnp.dot(es, wet_ref[...],
                 preferred_element_type=jnp.float32,
                 precision=lax.Precision.HIGHEST)
    out_ref[:, D_X_OUT:] = es + be_ref[...]


def _tc_dense(x, wxt, bx2, p0, p1, wet, be2):
    return pl.pallas_call(
        _tc_body,
        grid=(GRID_M,),
        in_specs=[
            pl.BlockSpec((ROW_BLOCK, D_X_IN), lambda i: (i, 0)),
            pl.BlockSpec((D_X_IN, D_X_OUT), lambda i: (0, 0)),
            pl.BlockSpec((1, D_X_OUT), lambda i: (0, 0)),
            pl.BlockSpec((ROW_BLOCK, D_E_IN), lambda i: (i, 0)),
            pl.BlockSpec((ROW_BLOCK, D_E_IN), lambda i: (i, 0)),
            pl.BlockSpec((D_E_IN, D_E_OUT), lambda i: (0, 0)),
            pl.BlockSpec((1, D_E_OUT), lambda i: (0, 0)),
        ],
        out_specs=pl.BlockSpec((ROW_BLOCK, D_X_OUT + D_E_OUT), lambda i: (i, 0)),
        out_shape=jax.ShapeDtypeStruct((N_NODES, D_X_OUT + D_E_OUT),
                                       jnp.float32),
    )(x, wxt, bx2, p0, p1, wet, be2)


def kernel(x, edge_index, edge_attr, Wx, bx, We, be):
    dst = edge_index[1].astype(jnp.int32)
    zeros = jnp.zeros((N_NODES, D_E_IN), jnp.float32)
    p0, p1 = _sc_segment_sum(edge_attr, dst, zeros)
    return _tc_dense(x, Wx.T, bx.reshape(1, -1), p0, p1, We.T,
                     be.reshape(1, -1))


# speedup vs baseline: 4.6579x; 1.3543x over previous
import functools

import jax
import jax.numpy as jnp
from jax import lax
from jax.experimental import pallas as pl
from jax.experimental.pallas import tpu as pltpu
from jax.experimental.pallas import tpu_sc as plsc

N_NODES = 10000
N_PAD = 10240
N_EDGES = 160000
D_X_IN = 256
D_X_OUT = 256
D_E_IN = 16
D_E_OUT = 64

NUM_SC = 2
EDGES_PER_CORE = N_EDGES // NUM_SC
CHUNK_E = 16000
NUM_CHUNKS = EDGES_PER_CORE // CHUNK_E
LANES = 16

_SC_PARAMS = pltpu.CompilerParams(
    use_tc_tiling_on_sc=False, needs_layout_passes=False)


def _sc_segment_sum_t(eaT, dst):
    mesh = plsc.VectorSubcoreMesh(core_axis_name="c", subcore_axis_name="s")
    part = jax.ShapeDtypeStruct((D_E_IN, N_PAD), jnp.float32)

    @functools.partial(
        pl.kernel,
        out_type=[part, part],
        mesh=mesh,
        compiler_params=_SC_PARAMS,
        scratch_types=[
            pltpu.VMEM((N_PAD,), jnp.float32),
            pltpu.VMEM((CHUNK_E,), jnp.int32),
            pltpu.VMEM((CHUNK_E,), jnp.float32),
            pltpu.VMEM((CHUNK_E,), jnp.int32),
            pltpu.VMEM((CHUNK_E,), jnp.float32),
            pltpu.SemaphoreType.DMA,
            pltpu.SemaphoreType.DMA,
        ],
    )
    def seg_sum(eaT_hbm, dst_hbm, p0_hbm, p1_hbm, acc,
                idx0, val0, idx1, val1, sem0, sem1):
        cid = lax.axis_index("c")
        sid = lax.axis_index("s")
        base = cid * EDGES_PER_CORE
        bufs = ((idx0, val0, sem0), (idx1, val1, sem1))

        def start(c, buf):
            idx_v, val_v, sem = buf
            off = base + c * CHUNK_E
            h1 = pltpu.async_copy(dst_hbm.at[pl.ds(off, CHUNK_E)], idx_v, sem)
            h2 = pltpu.async_copy(eaT_hbm.at[sid, pl.ds(off, CHUNK_E)],
                                  val_v, sem)
            return h1, h2

        pending = start(0, bufs[0])

        @pl.loop(0, N_PAD, step=LANES)
        def _(i):
            acc[pl.ds(i, LANES)] = jnp.zeros((LANES,), jnp.float32)

        for c in range(NUM_CHUNKS):
            idx_v, val_v, _ = bufs[c % 2]
            pending[0].wait()
            pending[1].wait()
            if c + 1 < NUM_CHUNKS:
                pending = start(c + 1, bufs[(c + 1) % 2])

            @pl.loop(0, CHUNK_E, step=4 * LANES)
            def _(i):
                for u in range(4):
                    o = i + u * LANES
                    plsc.addupdate_scatter(
                        acc, [idx_v[pl.ds(o, LANES)]], val_v[pl.ds(o, LANES)])

        @pl.when(cid == 0)
        def _():
            pltpu.sync_copy(acc, p0_hbm.at[sid])

        @pl.when(cid == 1)
        def _():
            pltpu.sync_copy(acc, p1_hbm.at[sid])

    return seg_sum(eaT, dst)


def _dst_body(ei_ref, dst_ref):
    dst_ref[...] = ei_ref[1, :]


def _extract_dst(edge_index):
    blk = 2048
    return pl.pallas_call(
        _dst_body,
        grid=(pl.cdiv(N_EDGES, blk),),
        in_specs=[pl.BlockSpec((2, blk), lambda i: (0, i))],
        out_specs=pl.BlockSpec((blk,), lambda i: (i,)),
        out_shape=jax.ShapeDtypeStruct((N_EDGES,), jnp.int32),
    )(edge_index)


ROW_BLOCK = 1024
GRID_M = pl.cdiv(N_NODES, ROW_BLOCK)


def _tc_body(x_ref, wxt_ref, bx_ref, p0_ref, p1_ref, wet_ref, be_ref, out_ref):
    xh = jnp.dot(x_ref[...], wxt_ref[...],
                 preferred_element_type=jnp.float32,
                 precision=lax.Precision.HIGHEST)
    out_ref[:, :D_X_OUT] = xh + bx_ref[...]
    s_t = p0_ref[...] + p1_ref[...]
    es = lax.dot_general(s_t, wet_ref[...], (((0,), (0,)), ((), ())),
                         preferred_element_type=jnp.float32,
                         precision=lax.Precision.HIGHEST)
    out_ref[:, D_X_OUT:] = es + be_ref[...]


def _tc_dense(x, wxt, bx2, p0t, p1t, wet, be2):
    return pl.pallas_call(
        _tc_body,
        grid=(GRID_M,),
        in_specs=[
            pl.BlockSpec((ROW_BLOCK, D_X_IN), lambda i: (i, 0)),
            pl.BlockSpec((D_X_IN, D_X_OUT), lambda i: (0, 0)),
            pl.BlockSpec((1, D_X_OUT), lambda i: (0, 0)),
            pl.BlockSpec((D_E_IN, ROW_BLOCK), lambda i: (0, i)),
            pl.BlockSpec((D_E_IN, ROW_BLOCK), lambda i: (0, i)),
            pl.BlockSpec((D_E_IN, D_E_OUT), lambda i: (0, 0)),
            pl.BlockSpec((1, D_E_OUT), lambda i: (0, 0)),
        ],
        out_specs=pl.BlockSpec((ROW_BLOCK, D_X_OUT + D_E_OUT), lambda i: (i, 0)),
        out_shape=jax.ShapeDtypeStruct((N_NODES, D_X_OUT + D_E_OUT),
                                       jnp.float32),
    )(x, wxt, bx2, p0t, p1t, wet, be2)


def kernel(x, edge_index, edge_attr, Wx, bx, We, be):
    dst = _extract_dst(edge_index.astype(jnp.int32))
    eaT = edge_attr.T
    p0t, p1t = _sc_segment_sum_t(eaT, dst)
    return _tc_dense(x, Wx.T, bx.reshape(1, -1), p0t, p1t, We.T,
                     be.reshape(1, -1))


# speedup vs baseline: 7.1480x; 1.5346x over previous
import functools

import jax
import jax.numpy as jnp
from jax import lax
from jax.experimental import pallas as pl
from jax.experimental.pallas import tpu as pltpu
from jax.experimental.pallas import tpu_sc as plsc

N_NODES = 10000
N_PAD = 10240
N_EDGES = 160000
D_X_IN = 256
D_X_OUT = 256
D_E_IN = 16
D_E_OUT = 64

NUM_SC = 2
EDGES_PER_CORE = N_EDGES // NUM_SC
CHUNK_E = 16000
NUM_CHUNKS = EDGES_PER_CORE // CHUNK_E
LANES = 16

_SC_PARAMS = pltpu.CompilerParams(
    use_tc_tiling_on_sc=False, needs_layout_passes=False)


def _sc_segment_sum_t(eaT, dst):
    mesh = plsc.VectorSubcoreMesh(core_axis_name="c", subcore_axis_name="s")
    part = jax.ShapeDtypeStruct((D_E_IN, N_PAD), jnp.float32)

    @functools.partial(
        pl.kernel,
        out_type=[part, part],
        mesh=mesh,
        compiler_params=_SC_PARAMS,
        scratch_types=[
            pltpu.VMEM((N_PAD,), jnp.float32),
            pltpu.VMEM((CHUNK_E,), jnp.int32),
            pltpu.VMEM((CHUNK_E,), jnp.float32),
            pltpu.VMEM((CHUNK_E,), jnp.int32),
            pltpu.VMEM((CHUNK_E,), jnp.float32),
            pltpu.SemaphoreType.DMA,
            pltpu.SemaphoreType.DMA,
        ],
    )
    def seg_sum(eaT_hbm, dst_hbm, p0_hbm, p1_hbm, acc,
                idx0, val0, idx1, val1, sem0, sem1):
        cid = lax.axis_index("c")
        sid = lax.axis_index("s")
        base = cid * EDGES_PER_CORE
        bufs = ((idx0, val0, sem0), (idx1, val1, sem1))

        def start(c, buf):
            idx_v, val_v, sem = buf
            off = base + c * CHUNK_E
            h1 = pltpu.async_copy(dst_hbm.at[pl.ds(off, CHUNK_E)], idx_v, sem)
            h2 = pltpu.async_copy(eaT_hbm.at[sid, pl.ds(off, CHUNK_E)],
                                  val_v, sem)
            return h1, h2

        pending = start(0, bufs[0])

        @pl.loop(0, N_PAD, step=LANES)
        def _(i):
            acc[pl.ds(i, LANES)] = jnp.zeros((LANES,), jnp.float32)

        for c in range(NUM_CHUNKS):
            idx_v, val_v, _ = bufs[c % 2]
            pending[0].wait()
            pending[1].wait()
            if c + 1 < NUM_CHUNKS:
                pending = start(c + 1, bufs[(c + 1) % 2])

            @plsc.parallel_loop(0, CHUNK_E, step=LANES, unroll=8)
            def _(i):
                plsc.addupdate_scatter(
                    acc, [idx_v[pl.ds(i, LANES)]], val_v[pl.ds(i, LANES)])

        @pl.when(cid == 0)
        def _():
            pltpu.sync_copy(acc, p0_hbm.at[sid])

        @pl.when(cid == 1)
        def _():
            pltpu.sync_copy(acc, p1_hbm.at[sid])

    return seg_sum(eaT, dst)


def _dst_body(ei_ref, dst_ref):
    dst_ref[...] = ei_ref[1, :]


def _extract_dst(edge_index):
    blk = 16384
    return pl.pallas_call(
        _dst_body,
        grid=(pl.cdiv(N_EDGES, blk),),
        in_specs=[pl.BlockSpec((2, blk), lambda i: (0, i))],
        out_specs=pl.BlockSpec((blk,), lambda i: (i,)),
        out_shape=jax.ShapeDtypeStruct((N_EDGES,), jnp.int32),
    )(edge_index)


ROW_BLOCK = 1024
GRID_M = pl.cdiv(N_NODES, ROW_BLOCK)


def _tc_body(x_ref, wxt_ref, bx_ref, p0_ref, p1_ref, wet_ref, be_ref, out_ref):
    xh = jnp.dot(x_ref[...], wxt_ref[...],
                 preferred_element_type=jnp.float32,
                 precision=lax.Precision.HIGHEST)
    out_ref[:, :D_X_OUT] = xh + bx_ref[...]
    s_t = p0_ref[...] + p1_ref[...]
    es = lax.dot_general(s_t, wet_ref[...], (((0,), (0,)), ((), ())),
                         preferred_element_type=jnp.float32,
                         precision=lax.Precision.HIGHEST)
    out_ref[:, D_X_OUT:] = es + be_ref[...]


def _tc_dense(x, wxt, bx2, p0t, p1t, wet, be2):
    return pl.pallas_call(
        _tc_body,
        grid=(GRID_M,),
        in_specs=[
            pl.BlockSpec((ROW_BLOCK, D_X_IN), lambda i: (i, 0)),
            pl.BlockSpec((D_X_IN, D_X_OUT), lambda i: (0, 0)),
            pl.BlockSpec((1, D_X_OUT), lambda i: (0, 0)),
            pl.BlockSpec((D_E_IN, ROW_BLOCK), lambda i: (0, i)),
            pl.BlockSpec((D_E_IN, ROW_BLOCK), lambda i: (0, i)),
            pl.BlockSpec((D_E_IN, D_E_OUT), lambda i: (0, 0)),
            pl.BlockSpec((1, D_E_OUT), lambda i: (0, 0)),
        ],
        out_specs=pl.BlockSpec((ROW_BLOCK, D_X_OUT + D_E_OUT), lambda i: (i, 0)),
        out_shape=jax.ShapeDtypeStruct((N_NODES, D_X_OUT + D_E_OUT),
                                       jnp.float32),
    )(x, wxt, bx2, p0t, p1t, wet, be2)


def kernel(x, edge_index, edge_attr, Wx, bx, We, be):
    dst = _extract_dst(edge_index.astype(jnp.int32))
    eaT = edge_attr.T
    p0t, p1t = _sc_segment_sum_t(eaT, dst)
    return _tc_dense(x, Wx.T, bx.reshape(1, -1), p0t, p1t, We.T,
                     be.reshape(1, -1))


# speedup vs baseline: 9.1286x; 1.2771x over previous
import functools

import jax
import jax.numpy as jnp
from jax import lax
from jax.experimental import pallas as pl
from jax.experimental.pallas import tpu as pltpu
from jax.experimental.pallas import tpu_sc as plsc

N_NODES = 10000
N_PAD = 10240
N_EDGES = 160000
D_X_IN = 256
D_X_OUT = 256
D_E_IN = 16
D_E_OUT = 64

NUM_SC = 2
EDGES_PER_CORE = N_EDGES // NUM_SC
CHUNK_E = 16000
NUM_CHUNKS = EDGES_PER_CORE // CHUNK_E
LANES = 16

_SC_PARAMS = pltpu.CompilerParams(
    use_tc_tiling_on_sc=False, needs_layout_passes=False)


def _sc_segment_sum_t(eaT, dst):
    mesh = plsc.VectorSubcoreMesh(core_axis_name="c", subcore_axis_name="s")
    part = jax.ShapeDtypeStruct((D_E_IN, N_PAD), jnp.float32)

    @functools.partial(
        pl.kernel,
        out_type=[part, part],
        mesh=mesh,
        compiler_params=_SC_PARAMS,
        scratch_types=[
            pltpu.VMEM((N_PAD,), jnp.float32),
            pltpu.VMEM((CHUNK_E,), jnp.int32),
            pltpu.VMEM((CHUNK_E,), jnp.float32),
            pltpu.VMEM((CHUNK_E,), jnp.int32),
            pltpu.VMEM((CHUNK_E,), jnp.float32),
            pltpu.SemaphoreType.DMA,
            pltpu.SemaphoreType.DMA,
        ],
    )
    def seg_sum(eaT_hbm, dst_hbm, p0_hbm, p1_hbm, acc,
                idx0, val0, idx1, val1, sem0, sem1):
        cid = lax.axis_index("c")
        sid = lax.axis_index("s")
        base = cid * EDGES_PER_CORE
        bufs = ((idx0, val0, sem0), (idx1, val1, sem1))

        def start(c, buf):
            idx_v, val_v, sem = buf
            off = base + c * CHUNK_E
            h1 = pltpu.async_copy(dst_hbm.at[pl.ds(off, CHUNK_E)], idx_v, sem)
            h2 = pltpu.async_copy(eaT_hbm.at[sid, pl.ds(off, CHUNK_E)],
                                  val_v, sem)
            return h1, h2

        pending = start(0, bufs[0])

        @pl.loop(0, N_PAD, step=LANES)
        def _(i):
            acc[pl.ds(i, LANES)] = jnp.zeros((LANES,), jnp.float32)

        for c in range(NUM_CHUNKS):
            idx_v, val_v, _ = bufs[c % 2]
            pending[0].wait()
            pending[1].wait()
            if c + 1 < NUM_CHUNKS:
                pending = start(c + 1, bufs[(c + 1) % 2])

            @plsc.parallel_loop(0, CHUNK_E, step=LANES, unroll=8)
            def _(i):
                plsc.addupdate_scatter(
                    acc, [idx_v[pl.ds(i, LANES)]], val_v[pl.ds(i, LANES)])

        @pl.when(cid == 0)
        def _():
            pltpu.sync_copy(acc, p0_hbm.at[sid])

        @pl.when(cid == 1)
        def _():
            pltpu.sync_copy(acc, p1_hbm.at[sid])

    return seg_sum(eaT, dst)


def _dst_body(ei_ref, dst_ref):
    dst_ref[...] = ei_ref[1, :]


def _extract_dst(edge_index):
    blk = 16384
    return pl.pallas_call(
        _dst_body,
        grid=(pl.cdiv(N_EDGES, blk),),
        in_specs=[pl.BlockSpec((2, blk), lambda i: (0, i))],
        out_specs=pl.BlockSpec((blk,), lambda i: (i,)),
        out_shape=jax.ShapeDtypeStruct((N_EDGES,), jnp.int32),
    )(edge_index)


ROW_BLOCK = 1024
GRID_M = pl.cdiv(N_NODES, ROW_BLOCK)


def _tc_body(x_ref, wx_ref, bxt_ref, p0_ref, p1_ref, we_ref, bet_ref,
             outt_ref):
    xht = lax.dot_general(wx_ref[...], x_ref[...], (((1,), (1,)), ((), ())),
                          preferred_element_type=jnp.float32)
    outt_ref[:D_X_OUT, :] = xht + bxt_ref[...]
    s_t = p0_ref[...] + p1_ref[...]
    est = lax.dot_general(we_ref[...], s_t, (((1,), (0,)), ((), ())),
                          preferred_element_type=jnp.float32)
    outt_ref[D_X_OUT:, :] = est + bet_ref[...]


def _tc_dense(x, wx, bxt, p0t, p1t, we, bet):
    outt = pl.pallas_call(
        _tc_body,
        grid=(GRID_M,),
        in_specs=[
            pl.BlockSpec((ROW_BLOCK, D_X_IN), lambda i: (i, 0)),
            pl.BlockSpec((D_X_OUT, D_X_IN), lambda i: (0, 0)),
            pl.BlockSpec((D_X_OUT, 1), lambda i: (0, 0)),
            pl.BlockSpec((D_E_IN, ROW_BLOCK), lambda i: (0, i)),
            pl.BlockSpec((D_E_IN, ROW_BLOCK), lambda i: (0, i)),
            pl.BlockSpec((D_E_OUT, D_E_IN), lambda i: (0, 0)),
            pl.BlockSpec((D_E_OUT, 1), lambda i: (0, 0)),
        ],
        out_specs=pl.BlockSpec((D_X_OUT + D_E_OUT, ROW_BLOCK),
                               lambda i: (0, i)),
        out_shape=jax.ShapeDtypeStruct((D_X_OUT + D_E_OUT, N_NODES),
                                       jnp.float32),
    )(x, wx, bxt, p0t, p1t, we, bet)
    return outt.T


def kernel(x, edge_index, edge_attr, Wx, bx, We, be):
    dst = _extract_dst(edge_index.astype(jnp.int32))
    eaT = edge_attr.T
    p0t, p1t = _sc_segment_sum_t(eaT, dst)
    return _tc_dense(x, Wx, bx.reshape(-1, 1), p0t, p1t, We,
                     be.reshape(-1, 1))


# speedup vs baseline: 9.4341x; 1.0335x over previous
import functools

import jax
import jax.numpy as jnp
from jax import lax
from jax.experimental import pallas as pl
from jax.experimental.pallas import tpu as pltpu
from jax.experimental.pallas import tpu_sc as plsc

N_NODES = 10000
N_PAD = 10240
N_EDGES = 160000
D_X_IN = 256
D_X_OUT = 256
D_E_IN = 16
D_E_OUT = 64

NUM_SC = 2
EDGES_PER_CORE = N_EDGES // NUM_SC
CHUNK_E = 16000
NUM_CHUNKS = EDGES_PER_CORE // CHUNK_E
LANES = 16

_SC_PARAMS = pltpu.CompilerParams(
    use_tc_tiling_on_sc=False, needs_layout_passes=False)


def _sc_segment_sum_t(eaT, dst):
    mesh = plsc.VectorSubcoreMesh(core_axis_name="c", subcore_axis_name="s")
    part = jax.ShapeDtypeStruct((D_E_IN, N_PAD), jnp.float32)

    @functools.partial(
        pl.kernel,
        out_type=[part, part],
        mesh=mesh,
        compiler_params=_SC_PARAMS,
        scratch_types=[
            pltpu.VMEM((N_PAD,), jnp.float32),
            pltpu.VMEM((CHUNK_E,), jnp.int32),
            pltpu.VMEM((CHUNK_E,), jnp.float32),
            pltpu.VMEM((CHUNK_E,), jnp.int32),
            pltpu.VMEM((CHUNK_E,), jnp.float32),
            pltpu.SemaphoreType.DMA,
            pltpu.SemaphoreType.DMA,
        ],
    )
    def seg_sum(eaT_hbm, dst_hbm, p0_hbm, p1_hbm, acc,
                idx0, val0, idx1, val1, sem0, sem1):
        cid = lax.axis_index("c")
        sid = lax.axis_index("s")
        base = cid * EDGES_PER_CORE
        bufs = ((idx0, val0, sem0), (idx1, val1, sem1))

        def start(c, buf):
            idx_v, val_v, sem = buf
            off = base + c * CHUNK_E
            h1 = pltpu.async_copy(dst_hbm.at[pl.ds(off, CHUNK_E)], idx_v, sem)
            h2 = pltpu.async_copy(eaT_hbm.at[sid, pl.ds(off, CHUNK_E)],
                                  val_v, sem)
            return h1, h2

        pending = start(0, bufs[0])

        @pl.loop(0, N_PAD, step=LANES)
        def _(i):
            acc[pl.ds(i, LANES)] = jnp.zeros((LANES,), jnp.float32)

        for c in range(NUM_CHUNKS):
            idx_v, val_v, _ = bufs[c % 2]
            pending[0].wait()
            pending[1].wait()
            if c + 1 < NUM_CHUNKS:
                pending = start(c + 1, bufs[(c + 1) % 2])

            @plsc.parallel_loop(0, CHUNK_E, step=LANES, unroll=8)
            def _(i):
                plsc.addupdate_scatter(
                    acc, [idx_v[pl.ds(i, LANES)]], val_v[pl.ds(i, LANES)])

        @pl.when(cid == 0)
        def _():
            pltpu.sync_copy(acc, p0_hbm.at[sid])

        @pl.when(cid == 1)
        def _():
            pltpu.sync_copy(acc, p1_hbm.at[sid])

    return seg_sum(eaT, dst)


def _dst_body(ei_ref, dst_ref):
    dst_ref[...] = ei_ref[1, :]


def _extract_dst(edge_index):
    blk = 16384
    return pl.pallas_call(
        _dst_body,
        grid=(pl.cdiv(N_EDGES, blk),),
        in_specs=[pl.BlockSpec((2, blk), lambda i: (0, i))],
        out_specs=pl.BlockSpec((blk,), lambda i: (i,)),
        out_shape=jax.ShapeDtypeStruct((N_EDGES,), jnp.int32),
    )(edge_index)


ROW_BLOCK = 1024
GRID_M = pl.cdiv(N_NODES, ROW_BLOCK)


def _tc_xh_body(x_ref, wx_ref, bxt_ref, outt_ref):
    xht = lax.dot_general(wx_ref[...], x_ref[...], (((1,), (1,)), ((), ())),
                          preferred_element_type=jnp.float32)
    outt_ref[...] = xht + bxt_ref[...]


def _tc_xh(x, wx, bxt):
    return pl.pallas_call(
        _tc_xh_body,
        grid=(GRID_M,),
        in_specs=[
            pl.BlockSpec((ROW_BLOCK, D_X_IN), lambda i: (i, 0)),
            pl.BlockSpec((D_X_OUT, D_X_IN), lambda i: (0, 0)),
            pl.BlockSpec((D_X_OUT, 1), lambda i: (0, 0)),
        ],
        out_specs=pl.BlockSpec((D_X_OUT, ROW_BLOCK), lambda i: (0, i)),
        out_shape=jax.ShapeDtypeStruct((D_X_OUT + D_E_OUT, N_NODES),
                                       jnp.float32),
    )(x, wx, bxt)


def _tc_es_body(outt_in_ref, p0_ref, p1_ref, we_ref, bet_ref, outt_ref):
    del outt_in_ref
    s_t = p0_ref[...] + p1_ref[...]
    est = lax.dot_general(we_ref[...], s_t, (((1,), (0,)), ((), ())),
                          preferred_element_type=jnp.float32)
    outt_ref[...] = est + bet_ref[...]


def _tc_es(outt, p0t, p1t, we, bet):
    return pl.pallas_call(
        _tc_es_body,
        grid=(GRID_M,),
        in_specs=[
            pl.BlockSpec(memory_space=pltpu.MemorySpace.HBM),
            pl.BlockSpec((D_E_IN, ROW_BLOCK), lambda i: (0, i)),
            pl.BlockSpec((D_E_IN, ROW_BLOCK), lambda i: (0, i)),
            pl.BlockSpec((D_E_OUT, D_E_IN), lambda i: (0, 0)),
            pl.BlockSpec((D_E_OUT, 1), lambda i: (0, 0)),
        ],
        out_specs=pl.BlockSpec((D_E_OUT, ROW_BLOCK),
                               lambda i: (D_X_OUT // D_E_OUT, i)),
        out_shape=jax.ShapeDtypeStruct((D_X_OUT + D_E_OUT, N_NODES),
                                       jnp.float32),
        input_output_aliases={0: 0},
    )(outt, p0t, p1t, we, bet)


def kernel(x, edge_index, edge_attr, Wx, bx, We, be):
    dst = _extract_dst(edge_index.astype(jnp.int32))
    eaT = edge_attr.T
    p0t, p1t = _sc_segment_sum_t(eaT, dst)
    outt = _tc_xh(x, Wx, bx.reshape(-1, 1))
    outt = _tc_es(outt, p0t, p1t, We, be.reshape(-1, 1))
    return outt.T


# speedup vs baseline: 11.2210x; 1.1894x over previous
import functools

import jax
import jax.numpy as jnp
from jax import lax
from jax.experimental import pallas as pl
from jax.experimental.pallas import tpu as pltpu
from jax.experimental.pallas import tpu_sc as plsc

N_NODES = 10000
N_PAD = 10240
N_EDGES = 160000
D_X_IN = 256
D_X_OUT = 256
D_E_IN = 16
D_E_OUT = 64

NUM_SC = 2
EDGES_PER_CORE = N_EDGES // NUM_SC
CHUNK_E = 16000
NUM_CHUNKS = EDGES_PER_CORE // CHUNK_E
LANES = 16

_SC_PARAMS = pltpu.CompilerParams(
    use_tc_tiling_on_sc=False, needs_layout_passes=False)


N_ROWS = N_PAD // 128


def _sc_segment_sum_t(eaT, edge_index):
    mesh = plsc.VectorSubcoreMesh(core_axis_name="c", subcore_axis_name="s")
    part = jax.ShapeDtypeStruct((D_E_IN, N_ROWS, 128), jnp.float32)

    @functools.partial(
        pl.kernel,
        out_type=[part, part],
        mesh=mesh,
        compiler_params=_SC_PARAMS,
        scratch_types=[
            pltpu.VMEM((N_ROWS, 128), jnp.float32),
            pltpu.VMEM((CHUNK_E,), jnp.int32),
            pltpu.VMEM((CHUNK_E,), jnp.float32),
            pltpu.VMEM((CHUNK_E,), jnp.int32),
            pltpu.VMEM((CHUNK_E,), jnp.float32),
            pltpu.SemaphoreType.DMA,
            pltpu.SemaphoreType.DMA,
        ],
    )
    def seg_sum(eaT_hbm, ei_hbm, p0_hbm, p1_hbm, acc,
                idx0, val0, idx1, val1, sem0, sem1):
        cid = lax.axis_index("c")
        sid = lax.axis_index("s")
        base = cid * EDGES_PER_CORE
        bufs = ((idx0, val0, sem0), (idx1, val1, sem1))

        def start(c, buf):
            idx_v, val_v, sem = buf
            off = base + c * CHUNK_E
            h1 = pltpu.async_copy(ei_hbm.at[1, pl.ds(off, CHUNK_E)], idx_v,
                                  sem)
            h2 = pltpu.async_copy(eaT_hbm.at[sid, pl.ds(off, CHUNK_E)],
                                  val_v, sem)
            return h1, h2

        pending = start(0, bufs[0])

        @pl.loop(0, N_ROWS)
        def _(r):
            @pl.loop(0, 128, step=LANES)
            def _(i):
                acc[r, pl.ds(i, LANES)] = jnp.zeros((LANES,), jnp.float32)

        for c in range(NUM_CHUNKS):
            idx_v, val_v, _ = bufs[c % 2]
            pending[0].wait()
            pending[1].wait()
            if c + 1 < NUM_CHUNKS:
                pending = start(c + 1, bufs[(c + 1) % 2])

            @plsc.parallel_loop(0, CHUNK_E, step=LANES, unroll=8)
            def _(i):
                idx = idx_v[pl.ds(i, LANES)]
                hi = lax.shift_right_logical(idx, 7)
                lo = lax.bitwise_and(idx, 127)
                plsc.addupdate_scatter(acc, [hi, lo], val_v[pl.ds(i, LANES)])

        @pl.when(cid == 0)
        def _():
            pltpu.sync_copy(acc, p0_hbm.at[sid])

        @pl.when(cid == 1)
        def _():
            pltpu.sync_copy(acc, p1_hbm.at[sid])

    return seg_sum(eaT, edge_index)


def _dst_body(ei_ref, dst_ref):
    dst_ref[...] = ei_ref[1, :]


def _extract_dst(edge_index):
    blk = 16384
    return pl.pallas_call(
        _dst_body,
        grid=(pl.cdiv(N_EDGES, blk),),
        in_specs=[pl.BlockSpec((2, blk), lambda i: (0, i))],
        out_specs=pl.BlockSpec((blk,), lambda i: (i,)),
        out_shape=jax.ShapeDtypeStruct((N_EDGES,), jnp.int32),
    )(edge_index)


ROW_BLOCK = 1024
GRID_M = pl.cdiv(N_NODES, ROW_BLOCK)


def _tc_xh_body(x_ref, wx_ref, bxt_ref, outt_ref):
    xht = lax.dot_general(wx_ref[...], x_ref[...], (((1,), (1,)), ((), ())),
                          preferred_element_type=jnp.float32)
    outt_ref[...] = xht + bxt_ref[...]


def _tc_xh(x, wx, bxt):
    return pl.pallas_call(
        _tc_xh_body,
        grid=(GRID_M,),
        in_specs=[
            pl.BlockSpec((ROW_BLOCK, D_X_IN), lambda i: (i, 0)),
            pl.BlockSpec((D_X_OUT, D_X_IN), lambda i: (0, 0)),
            pl.BlockSpec((D_X_OUT, 1), lambda i: (0, 0)),
        ],
        out_specs=pl.BlockSpec((D_X_OUT, ROW_BLOCK), lambda i: (0, i)),
        out_shape=jax.ShapeDtypeStruct((D_X_OUT + D_E_OUT, N_NODES),
                                       jnp.float32),
    )(x, wx, bxt)


def _tc_es_body(outt_in_ref, p0_ref, p1_ref, we_ref, bet_ref, outt_ref):
    del outt_in_ref
    s_t = jnp.reshape(p0_ref[...] + p1_ref[...], (D_E_IN, N_PAD))
    est = lax.dot_general(we_ref[...], s_t[:, :N_NODES],
                          (((1,), (0,)), ((), ())),
                          preferred_element_type=jnp.float32)
    outt_ref[...] = est + bet_ref[...]


def _tc_es(outt, p0t, p1t, we, bet):
    return pl.pallas_call(
        _tc_es_body,
        grid=(1,),
        in_specs=[
            pl.BlockSpec(memory_space=pltpu.MemorySpace.HBM),
            pl.BlockSpec((D_E_IN, N_ROWS, 128), lambda i: (0, 0, 0)),
            pl.BlockSpec((D_E_IN, N_ROWS, 128), lambda i: (0, 0, 0)),
            pl.BlockSpec((D_E_OUT, D_E_IN), lambda i: (0, 0)),
            pl.BlockSpec((D_E_OUT, 1), lambda i: (0, 0)),
        ],
        out_specs=pl.BlockSpec((D_E_OUT, N_NODES),
                               lambda i: (D_X_OUT // D_E_OUT, 0)),
        out_shape=jax.ShapeDtypeStruct((D_X_OUT + D_E_OUT, N_NODES),
                                       jnp.float32),
        input_output_aliases={0: 0},
    )(outt, p0t, p1t, we, bet)


def kernel(x, edge_index, edge_attr, Wx, bx, We, be):
    eaT = edge_attr.T
    p0t, p1t = _sc_segment_sum_t(eaT, edge_index.astype(jnp.int32))
    outt = _tc_xh(x, Wx, bx.reshape(-1, 1))
    outt = _tc_es(outt, p0t, p1t, We, be.reshape(-1, 1))
    return outt.T
